# Initial kernel scaffold; baseline (speedup 1.0000x reference)
import jax
import jax.numpy as jnp
from jax.experimental import pallas as pl


def kernel(z_compound, z_protein, edge_label_index, attn_w, attn_b, lin1_w, lin1_b, lin2_w, lin2_b):
    raise NotImplementedError("write your pallas kernel here")


# speedup vs baseline: 1.6896x; 1.6896x over previous
import functools

import jax
import jax.numpy as jnp
from jax import lax
from jax.experimental import pallas as pl
from jax.experimental.pallas import tpu as pltpu
from jax.experimental.pallas import tpu_sc as plsc

H = 64
N_NODES = 50000
N_PAD = 50048
NC = 2
NS = 16
NW = NC * NS
E_TOTAL = 800000
EPW = 25088
E_PAD = NW * EPW
C1 = 3136
NCH1 = EPW // C1
C2 = 128
NCH2 = EPW // C2
SENTINEL = -1.0e30

_mesh = plsc.VectorSubcoreMesh(core_axis_name="c", subcore_axis_name="s")
_sc_params = pltpu.CompilerParams(needs_layout_passes=False,
                                  use_tc_tiling_on_sc=False)


def _worker_id():
    return lax.axis_index("s") * NC + lax.axis_index("c")


def _precompute_body(zc_ref, zp_ref, w1t_ref, wa_ref, u_ref, v_ref, ac_ref, ap_ref):
    zc = zc_ref[...]
    zp = zp_ref[...]
    w1t = w1t_ref[...]
    wa = wa_ref[...]
    u_ref[...] = jnp.dot(zc, w1t[:H, :], preferred_element_type=jnp.float32)
    v_ref[...] = jnp.dot(zp, w1t[H:, :], preferred_element_type=jnp.float32)
    ac_ref[...] = jnp.sum(zc * wa[0, :H][None, :], axis=1, keepdims=True)
    ap_ref[...] = jnp.sum(zp * wa[0, H:][None, :], axis=1, keepdims=True)


def _precompute(z_c, z_p, w1t, attn_w):
    R = 512
    grid = (pl.cdiv(N_PAD, R),)
    return pl.pallas_call(
        _precompute_body,
        grid=grid,
        in_specs=[
            pl.BlockSpec((R, H), lambda i: (i, 0)),
            pl.BlockSpec((R, H), lambda i: (i, 0)),
            pl.BlockSpec((2 * H, H), lambda i: (0, 0)),
            pl.BlockSpec((1, 2 * H), lambda i: (0, 0)),
        ],
        out_specs=[
            pl.BlockSpec((R, H), lambda i: (i, 0)),
            pl.BlockSpec((R, H), lambda i: (i, 0)),
            pl.BlockSpec((R, 1), lambda i: (i, 0)),
            pl.BlockSpec((R, 1), lambda i: (i, 0)),
        ],
        out_shape=[
            jax.ShapeDtypeStruct((N_PAD, H), jnp.float32),
            jax.ShapeDtypeStruct((N_PAD, H), jnp.float32),
            jax.ShapeDtypeStruct((N_NODES, 1), jnp.float32),
            jax.ShapeDtypeStruct((N_NODES, 1), jnp.float32),
        ],
    )(z_c, z_p, w1t, attn_w)


def _logits_body(src_h, dst_h, ac_h, ap_h, lg_h, tmax_h, tsum_h,
                 acv, apv, srcv, dstv, lgv, statv):
    wid = _worker_id()
    base = wid * EPW
    pltpu.sync_copy(ac_h, acv)
    pltpu.sync_copy(ap_h, apv)

    def chunk_body(j, carry):
        m, s = carry
        off = base + j * C1
        pltpu.sync_copy(src_h.at[pl.ds(off, C1)], srcv)
        pltpu.sync_copy(dst_h.at[pl.ds(off, C1)], dstv)

        def step(t, carry2):
            m, s = carry2
            si = srcv[pl.ds(t * 16, 16)]
            di = dstv[pl.ds(t * 16, 16)]
            l = plsc.load_gather(acv, [si]) + plsc.load_gather(apv, [di])
            lgv[pl.ds(t * 16, 16)] = l
            mn = jnp.maximum(m, l)
            s = s * jnp.exp(m - mn) + jnp.exp(l - mn)
            return mn, s

        m, s = lax.fori_loop(0, C1 // 16, step, (m, s))
        pltpu.sync_copy(lgv, lg_h.at[pl.ds(off, C1)])
        return m, s

    m0 = jnp.full((16,), SENTINEL, jnp.float32)
    s0 = jnp.zeros((16,), jnp.float32)
    m, s = lax.fori_loop(0, NCH1, chunk_body, (m0, s0))
    statv[pl.ds(0, 16)] = m
    pltpu.sync_copy(statv, tmax_h.at[pl.ds(wid * 16, 16)])
    statv[pl.ds(0, 16)] = s
    pltpu.sync_copy(statv, tsum_h.at[pl.ds(wid * 16, 16)])


_logits_kernel = pl.kernel(
    _logits_body,
    out_type=(
        jax.ShapeDtypeStruct((E_PAD,), jnp.float32),
        jax.ShapeDtypeStruct((NW * 16,), jnp.float32),
        jax.ShapeDtypeStruct((NW * 16,), jnp.float32),
    ),
    mesh=_mesh,
    compiler_params=_sc_params,
    scratch_types=[
        pltpu.VMEM((N_PAD,), jnp.float32),
        pltpu.VMEM((N_PAD,), jnp.float32),
        pltpu.VMEM((C1,), jnp.int32),
        pltpu.VMEM((C1,), jnp.int32),
        pltpu.VMEM((C1,), jnp.float32),
        pltpu.VMEM((16,), jnp.float32),
    ],
)


def _decode_body(src_h, dst_h, lg_h, tmax_h, tsum_h, u_h, v_h, c_h, out_h,
                 srcv, dstv, lgv, uv, vv, outv, cv, mxv, smv, sem):
    wid = _worker_id()
    base = wid * EPW
    pltpu.sync_copy(c_h, cv)
    pltpu.sync_copy(tmax_h, mxv)
    pltpu.sync_copy(tsum_h, smv)

    m = jnp.full((16,), SENTINEL, jnp.float32)
    for i in range(NW):
        m = jnp.maximum(m, mxv[pl.ds(i * 16, 16)])
    M = jnp.max(m)
    sacc = jnp.zeros((16,), jnp.float32)
    for i in range(NW):
        sacc = sacc + smv[pl.ds(i * 16, 16)] * jnp.exp(mxv[pl.ds(i * 16, 16)] - M)
    S = jnp.sum(sacc)
    invS = (jnp.ones((16,), jnp.float32) / jnp.full((16,), S))[0]

    b2 = cv[pl.ds(2 * H, 16)][0]

    def chunk(j, _):
        off = base + j * C2
        pltpu.sync_copy(src_h.at[pl.ds(off, C2)], srcv)
        pltpu.sync_copy(dst_h.at[pl.ds(off, C2)], dstv)
        pltpu.sync_copy(lg_h.at[pl.ds(off, C2)], lgv)
        pltpu.async_copy(u_h.at[srcv], uv, sem).wait()
        pltpu.async_copy(v_h.at[dstv], vv, sem).wait()

        def grp(g, _g):
            e16 = lax.iota(jnp.int32, 16) + g * 16
            sc = jnp.exp(lgv[pl.ds(g * 16, 16)] - M) * invS
            acc0 = jnp.zeros((16,), jnp.float32)
            acc1 = jnp.zeros((16,), jnp.float32)
            acc2 = jnp.zeros((16,), jnp.float32)
            acc3 = jnp.zeros((16,), jnp.float32)
            accs = [acc0, acc1, acc2, acc3]
            for fg in range(H // 16):
                b1v = cv[pl.ds(fg * 16, 16)]
                w2v = cv[pl.ds(H + fg * 16, 16)]
                for k in range(16):
                    f = fg * 16 + k
                    colf = jnp.full((16,), f, jnp.int32)
                    uf = plsc.load_gather(uv, [e16, colf])
                    vf = plsc.load_gather(vv, [e16, colf])
                    t = (uf + vf) * sc + b1v[k]
                    r = jnp.maximum(t, 0.0)
                    accs[f % 4] = accs[f % 4] + r * w2v[k]
            o = (accs[0] + accs[1]) + (accs[2] + accs[3]) + b2
            o = 1.0 / (1.0 + jnp.exp(-o))
            outv[pl.ds(g * 16, 16)] = o
            return 0

        lax.fori_loop(0, C2 // 16, grp, 0)
        pltpu.sync_copy(outv, out_h.at[pl.ds(off, C2)])
        return 0

    lax.fori_loop(0, NCH2, chunk, 0)


_decode_kernel = pl.kernel(
    _decode_body,
    out_type=jax.ShapeDtypeStruct((E_PAD,), jnp.float32),
    mesh=_mesh,
    compiler_params=_sc_params,
    scratch_types=[
        pltpu.VMEM((C2,), jnp.int32),
        pltpu.VMEM((C2,), jnp.int32),
        pltpu.VMEM((C2,), jnp.float32),
        pltpu.VMEM((C2, H), jnp.float32),
        pltpu.VMEM((C2, H), jnp.float32),
        pltpu.VMEM((C2,), jnp.float32),
        pltpu.VMEM((2 * H + 16,), jnp.float32),
        pltpu.VMEM((NW * 16,), jnp.float32),
        pltpu.VMEM((NW * 16,), jnp.float32),
        pltpu.SemaphoreType.DMA,
    ],
)


def kernel(z_compound, z_protein, edge_label_index, attn_w, attn_b,
           lin1_w, lin1_b, lin2_w, lin2_b):
    del attn_b
    f32 = jnp.float32
    w1t = lin1_w.T.astype(f32)
    U, V, ac, ap = _precompute(z_compound, z_protein, w1t, attn_w)

    pad = E_PAD - E_TOTAL
    src = jnp.concatenate([edge_label_index[0].astype(jnp.int32),
                           jnp.full((pad,), N_NODES, jnp.int32)])
    dst = jnp.concatenate([edge_label_index[1].astype(jnp.int32),
                           jnp.full((pad,), N_NODES, jnp.int32)])
    sent = jnp.full((N_PAD - N_NODES,), SENTINEL, f32)
    ac_t = jnp.concatenate([ac.reshape(-1), sent])
    ap_t = jnp.concatenate([ap.reshape(-1), sent])

    logits, tmax, tsum = _logits_kernel(src, dst, ac_t, ap_t)

    consts = jnp.concatenate([lin1_b.astype(f32), lin2_w.reshape(-1).astype(f32),
                              jnp.broadcast_to(lin2_b.astype(f32), (16,))])
    out = _decode_kernel(src, dst, logits, tmax, tsum, U, V, consts)
    return out[:E_TOTAL]


# speedup vs baseline: 2.3800x; 1.4086x over previous
import functools

import jax
import jax.numpy as jnp
from jax import lax
from jax.experimental import pallas as pl
from jax.experimental.pallas import tpu as pltpu
from jax.experimental.pallas import tpu_sc as plsc

H = 64
N_NODES = 50000
N_PAD = 50048
NC = 2
NS = 16
NW = NC * NS
E_TOTAL = 800000
EPW = 25088
E_PAD = NW * EPW
C1 = 3136
NCH1 = EPW // C1
C2 = 128
NCH2 = EPW // C2
SENTINEL = -1.0e30

_mesh = plsc.VectorSubcoreMesh(core_axis_name="c", subcore_axis_name="s")
_sc_params = pltpu.CompilerParams(needs_layout_passes=False,
                                  use_tc_tiling_on_sc=False)


def _worker_id():
    return lax.axis_index("s") * NC + lax.axis_index("c")


def _precompute_body(zc_ref, zp_ref, w1t_ref, wa_ref, u_ref, v_ref, ac_ref, ap_ref):
    zc = zc_ref[...]
    zp = zp_ref[...]
    w1t = w1t_ref[...]
    wa = wa_ref[...]
    u_ref[...] = jnp.dot(zc, w1t[:H, :], preferred_element_type=jnp.float32)
    v_ref[...] = jnp.dot(zp, w1t[H:, :], preferred_element_type=jnp.float32)
    ac_ref[...] = jnp.sum(zc * wa[0, :H][None, :], axis=1, keepdims=True)
    ap_ref[...] = jnp.sum(zp * wa[0, H:][None, :], axis=1, keepdims=True)


def _precompute(z_c, z_p, w1t, attn_w):
    R = 512
    grid = (pl.cdiv(N_PAD, R),)
    return pl.pallas_call(
        _precompute_body,
        grid=grid,
        in_specs=[
            pl.BlockSpec((R, H), lambda i: (i, 0)),
            pl.BlockSpec((R, H), lambda i: (i, 0)),
            pl.BlockSpec((2 * H, H), lambda i: (0, 0)),
            pl.BlockSpec((1, 2 * H), lambda i: (0, 0)),
        ],
        out_specs=[
            pl.BlockSpec((R, H), lambda i: (i, 0)),
            pl.BlockSpec((R, H), lambda i: (i, 0)),
            pl.BlockSpec((R, 1), lambda i: (i, 0)),
            pl.BlockSpec((R, 1), lambda i: (i, 0)),
        ],
        out_shape=[
            jax.ShapeDtypeStruct((N_PAD, H), jnp.float32),
            jax.ShapeDtypeStruct((N_PAD, H), jnp.float32),
            jax.ShapeDtypeStruct((N_NODES, 1), jnp.float32),
            jax.ShapeDtypeStruct((N_NODES, 1), jnp.float32),
        ],
    )(z_c, z_p, w1t, attn_w)


def _logits_body(src_h, dst_h, ac_h, ap_h, lg_h, tmax_h, tsum_h,
                 acv, apv, srcv, dstv, lgv, statv):
    wid = _worker_id()
    base = wid * EPW
    pltpu.sync_copy(ac_h, acv)
    pltpu.sync_copy(ap_h, apv)

    def chunk_body(j, carry):
        m, s = carry
        off = base + j * C1
        pltpu.sync_copy(src_h.at[pl.ds(off, C1)], srcv)
        pltpu.sync_copy(dst_h.at[pl.ds(off, C1)], dstv)

        def step(t, carry2):
            m, s = carry2
            si = srcv[pl.ds(t * 16, 16)]
            di = dstv[pl.ds(t * 16, 16)]
            l = plsc.load_gather(acv, [si]) + plsc.load_gather(apv, [di])
            lgv[pl.ds(t * 16, 16)] = l
            mn = jnp.maximum(m, l)
            s = s * jnp.exp(m - mn) + jnp.exp(l - mn)
            return mn, s

        m, s = lax.fori_loop(0, C1 // 16, step, (m, s))
        pltpu.sync_copy(lgv, lg_h.at[pl.ds(off, C1)])
        return m, s

    m0 = jnp.full((16,), SENTINEL, jnp.float32)
    s0 = jnp.zeros((16,), jnp.float32)
    m, s = lax.fori_loop(0, NCH1, chunk_body, (m0, s0))
    statv[pl.ds(0, 16)] = m
    pltpu.sync_copy(statv, tmax_h.at[pl.ds(wid * 16, 16)])
    statv[pl.ds(0, 16)] = s
    pltpu.sync_copy(statv, tsum_h.at[pl.ds(wid * 16, 16)])


_logits_kernel = pl.kernel(
    _logits_body,
    out_type=(
        jax.ShapeDtypeStruct((E_PAD,), jnp.float32),
        jax.ShapeDtypeStruct((NW * 16,), jnp.float32),
        jax.ShapeDtypeStruct((NW * 16,), jnp.float32),
    ),
    mesh=_mesh,
    compiler_params=_sc_params,
    scratch_types=[
        pltpu.VMEM((N_PAD,), jnp.float32),
        pltpu.VMEM((N_PAD,), jnp.float32),
        pltpu.VMEM((C1,), jnp.int32),
        pltpu.VMEM((C1,), jnp.int32),
        pltpu.VMEM((C1,), jnp.float32),
        pltpu.VMEM((16,), jnp.float32),
    ],
)


def _decode_body(src_h, dst_h, lg_h, tmax_h, tsum_h, u_h, v_h, c_h, out_h,
                 src0, dst0, lg0, uv0, vv0, ov0,
                 src1, dst1, lg1, uv1, vv1, ov1,
                 cv, mxv, smv,
                 semi0, semi1, semg0, semg1, semo0, semo1):
    wid = _worker_id()
    base = wid * EPW
    pltpu.sync_copy(c_h, cv)
    pltpu.sync_copy(tmax_h, mxv)
    pltpu.sync_copy(tsum_h, smv)

    bufs = ((src0, dst0, lg0, uv0, vv0, ov0, semi0, semg0, semo0),
            (src1, dst1, lg1, uv1, vv1, ov1, semi1, semg1, semo1))

    m = jnp.full((16,), SENTINEL, jnp.float32)
    for i in range(NW):
        m = jnp.maximum(m, mxv[pl.ds(i * 16, 16)])
    M = jnp.max(m)
    sacc = jnp.zeros((16,), jnp.float32)
    for i in range(NW):
        sacc = sacc + smv[pl.ds(i * 16, 16)] * jnp.exp(mxv[pl.ds(i * 16, 16)] - M)
    S = jnp.sum(sacc)
    invS = (jnp.ones((16,), jnp.float32) / jnp.full((16,), S))[0]

    b2 = cv[pl.ds(2 * H, 16)][0]

    def idx_copies(j, b):
        off = base + j * C2
        sv, dv, lv = bufs[b][0], bufs[b][1], bufs[b][2]
        semi = bufs[b][6]
        return (pltpu.make_async_copy(src_h.at[pl.ds(off, C2)], sv, semi),
                pltpu.make_async_copy(dst_h.at[pl.ds(off, C2)], dv, semi),
                pltpu.make_async_copy(lg_h.at[pl.ds(off, C2)], lv, semi))

    def gather_copies(b):
        sv, dv, uv, vv = bufs[b][0], bufs[b][1], bufs[b][3], bufs[b][4]
        semg = bufs[b][7]
        return (pltpu.make_async_copy(u_h.at[sv], uv, semg),
                pltpu.make_async_copy(v_h.at[dv], vv, semg))

    def out_copy(j, b):
        off = base + j * C2
        return pltpu.make_async_copy(bufs[b][5], out_h.at[pl.ds(off, C2)],
                                     bufs[b][8])

    def start_all(descs):
        for d in descs:
            d.start()

    def wait_all(descs):
        for d in descs:
            d.wait()

    def compute(b):
        lv, uv, vv, ov = bufs[b][2], bufs[b][3], bufs[b][4], bufs[b][5]

        def grp(g, _g):
            e16 = lax.iota(jnp.int32, 16) + g * 16
            sc = jnp.exp(lv[pl.ds(g * 16, 16)] - M) * invS
            accs = [jnp.zeros((16,), jnp.float32) for _ in range(4)]
            for fg in range(H // 16):
                b1v = cv[pl.ds(fg * 16, 16)]
                w2v = cv[pl.ds(H + fg * 16, 16)]
                for k in range(16):
                    f = fg * 16 + k
                    colf = jnp.full((16,), f, jnp.int32)
                    uf = plsc.load_gather(uv, [e16, colf])
                    vf = plsc.load_gather(vv, [e16, colf])
                    t = (uf + vf) * sc + b1v[k]
                    r = jnp.maximum(t, 0.0)
                    accs[f % 4] = accs[f % 4] + r * w2v[k]
            o = (accs[0] + accs[1]) + (accs[2] + accs[3]) + b2
            o = 1.0 / (1.0 + jnp.exp(-o))
            ov[pl.ds(g * 16, 16)] = o
            return 0

        lax.fori_loop(0, C2 // 16, grp, 0)

    start_all(idx_copies(0, 0))
    start_all(idx_copies(1, 1))
    wait_all(idx_copies(0, 0))
    start_all(gather_copies(0))

    def pair(jj, _):
        for b in (0, 1):
            j = 2 * jj + b
            nb = 1 - b

            @pl.when(j + 1 < NCH2)
            def _():
                wait_all(idx_copies(j + 1, nb))
                start_all(gather_copies(nb))

            wait_all(gather_copies(b))

            @pl.when(j + 2 < NCH2)
            def _():
                start_all(idx_copies(j + 2, b))

            @pl.when(j >= 2)
            def _():
                out_copy(j - 2, b).wait()

            compute(b)
            out_copy(j, b).start()
        return 0

    lax.fori_loop(0, NCH2 // 2, pair, 0)
    out_copy(NCH2 - 2, 0).wait()
    out_copy(NCH2 - 1, 1).wait()


_decode_kernel = pl.kernel(
    _decode_body,
    out_type=jax.ShapeDtypeStruct((E_PAD,), jnp.float32),
    mesh=_mesh,
    compiler_params=_sc_params,
    scratch_types=[
        pltpu.VMEM((C2,), jnp.int32),
        pltpu.VMEM((C2,), jnp.int32),
        pltpu.VMEM((C2,), jnp.float32),
        pltpu.VMEM((C2, H), jnp.float32),
        pltpu.VMEM((C2, H), jnp.float32),
        pltpu.VMEM((C2,), jnp.float32),
        pltpu.VMEM((C2,), jnp.int32),
        pltpu.VMEM((C2,), jnp.int32),
        pltpu.VMEM((C2,), jnp.float32),
        pltpu.VMEM((C2, H), jnp.float32),
        pltpu.VMEM((C2, H), jnp.float32),
        pltpu.VMEM((C2,), jnp.float32),
        pltpu.VMEM((2 * H + 16,), jnp.float32),
        pltpu.VMEM((NW * 16,), jnp.float32),
        pltpu.VMEM((NW * 16,), jnp.float32),
        pltpu.SemaphoreType.DMA,
        pltpu.SemaphoreType.DMA,
        pltpu.SemaphoreType.DMA,
        pltpu.SemaphoreType.DMA,
        pltpu.SemaphoreType.DMA,
        pltpu.SemaphoreType.DMA,
    ],
)


def kernel(z_compound, z_protein, edge_label_index, attn_w, attn_b,
           lin1_w, lin1_b, lin2_w, lin2_b):
    del attn_b
    f32 = jnp.float32
    w1t = lin1_w.T.astype(f32)
    U, V, ac, ap = _precompute(z_compound, z_protein, w1t, attn_w)

    pad = E_PAD - E_TOTAL
    src = jnp.concatenate([edge_label_index[0].astype(jnp.int32),
                           jnp.full((pad,), N_NODES, jnp.int32)])
    dst = jnp.concatenate([edge_label_index[1].astype(jnp.int32),
                           jnp.full((pad,), N_NODES, jnp.int32)])
    sent = jnp.full((N_PAD - N_NODES,), SENTINEL, f32)
    ac_t = jnp.concatenate([ac.reshape(-1), sent])
    ap_t = jnp.concatenate([ap.reshape(-1), sent])

    logits, tmax, tsum = _logits_kernel(src, dst, ac_t, ap_t)

    consts = jnp.concatenate([lin1_b.astype(f32), lin2_w.reshape(-1).astype(f32),
                              jnp.broadcast_to(lin2_b.astype(f32), (16,))])
    out = _decode_kernel(src, dst, logits, tmax, tsum, U, V, consts)
    return out[:E_TOTAL]
